# Initial kernel scaffold; baseline (speedup 1.0000x reference)
import jax
import jax.numpy as jnp
from jax.experimental import pallas as pl


def kernel(x, edge_index, W1, b1, W2, b2, W3, b3):
    raise NotImplementedError("write your pallas kernel here")


# speedup vs baseline: 7.0221x; 7.0221x over previous
import functools

import jax
import jax.numpy as jnp
from jax import lax
from jax.experimental import pallas as pl
from jax.experimental.pallas import tpu as pltpu
from jax.experimental.pallas import tpu_sc as plsc

N = 10000
E = 320000
IN_C = 128
HID_C = 128
OUT_C = 64

NC, NS = 2, 16
NT = NC * NS
EB = 128
EPT = 10240
E_PAD = NT * EPT
NB = EPT // EB
N_PAD = 10240
RPT = N_PAD // NS
ZCH = 64

_MESH = plsc.VectorSubcoreMesh(
    core_axis_name="c", subcore_axis_name="s", num_cores=NC, num_subcores=NS
)


def _make_agg(C):
    @functools.partial(
        pl.kernel,
        out_type=jax.ShapeDtypeStruct((NC, N_PAD, C), jnp.float32),
        mesh=_MESH,
        scratch_types=[
            pltpu.VMEM((EB,), jnp.int32),
            pltpu.VMEM((EB,), jnp.int32),
            pltpu.VMEM((EB, C), jnp.float32),
            pltpu.VMEM((ZCH, C), jnp.float32),
            pltpu.VMEM_SHARED((N_PAD, C), jnp.float32),
            pltpu.SemaphoreType.DMA,
        ],
        compiler_params=pltpu.CompilerParams(use_tc_tiling_on_sc=False),
    )
    def agg(hs, src, dst, out, srci, dsti, rows, chunk, acc, sem):
        cid = lax.axis_index("c")
        sid = lax.axis_index("s")

        def zlane(t, _):
            chunk[t // (C // 16), pl.ds((t % (C // 16)) * 16, 16)] = jnp.zeros(
                (16,), jnp.float32
            )
            return _

        lax.fori_loop(0, ZCH * (C // 16), zlane, None)
        r0 = sid * RPT

        def zchunk(k, _):
            pltpu.sync_copy(chunk, acc.at[pl.ds(r0 + k * ZCH, ZCH)])
            return _

        lax.fori_loop(0, RPT // ZCH, zchunk, None)
        plsc.subcore_barrier()

        e0 = (cid * NS + sid) * EPT

        def batch(b, _):
            base = e0 + b * EB
            pltpu.sync_copy(src.at[pl.ds(base, EB)], srci)
            pltpu.sync_copy(dst.at[pl.ds(base, EB)], dsti)
            pltpu.async_copy(hs.at[srci], rows, sem).wait()
            pltpu.sync_copy(rows, acc.at[dsti], add=True)
            return _

        lax.fori_loop(0, NB, batch, None)
        plsc.subcore_barrier()

        def cpout(k, _):
            pltpu.sync_copy(acc.at[pl.ds(r0 + k * ZCH, ZCH)], chunk)
            pltpu.sync_copy(chunk, out.at[cid].at[pl.ds(r0 + k * ZCH, ZCH)])
            return _

        lax.fori_loop(0, RPT // ZCH, cpout, None)

    return agg


_agg16 = _make_agg(16)
_agg128 = _make_agg(HID_C)
_agg64 = _make_agg(OUT_C)

_BM = 1000
_GRID = (N // _BM,)


def _prep_body(degp_ref, x_ref, w_ref, dinv_ref, hs_ref):
    deg = degp_ref[0, :, 0] + degp_ref[1, :, 0] + 1.0
    dv = lax.rsqrt(deg)[:, None]
    dinv_ref[...] = dv
    hs_ref[...] = jnp.dot(x_ref[...], w_ref[...], preferred_element_type=jnp.float32) * dv


def _mid_body(p_ref, hs_ref, dinv_ref, b_ref, w_ref, o_ref):
    dv = dinv_ref[...]
    t = (p_ref[0] + p_ref[1] + hs_ref[...]) * dv + b_ref[...]
    a = jnp.maximum(t, 0.0)
    o_ref[...] = jnp.dot(a, w_ref[...], preferred_element_type=jnp.float32) * dv


def _final_body(p_ref, hs_ref, dinv_ref, b_ref, o_ref):
    t = (p_ref[0] + p_ref[1] + hs_ref[...]) * dinv_ref[...] + b_ref[...]
    m = jnp.max(t, axis=1, keepdims=True)
    lse = jnp.log(jnp.sum(jnp.exp(t - m), axis=1, keepdims=True)) + m
    o_ref[...] = t - lse


def _row_spec(c):
    return pl.BlockSpec((_BM, c), lambda i: (i, 0))


def _p_spec(c):
    return pl.BlockSpec((NC, _BM, c), lambda i: (0, i, 0))


def _full_spec(a, b):
    return pl.BlockSpec((a, b), lambda i: (0, 0))


_prep = pl.pallas_call(
    _prep_body,
    grid=_GRID,
    in_specs=[_p_spec(16), _row_spec(IN_C), _full_spec(IN_C, HID_C)],
    out_specs=[_row_spec(1), _row_spec(HID_C)],
    out_shape=[
        jax.ShapeDtypeStruct((N, 1), jnp.float32),
        jax.ShapeDtypeStruct((N, HID_C), jnp.float32),
    ],
)

def _mid(cin, cout):
    return pl.pallas_call(
        _mid_body,
        grid=_GRID,
        in_specs=[
            _p_spec(cin),
            _row_spec(cin),
            _row_spec(1),
            _full_spec(1, cin),
            _full_spec(cin, cout),
        ],
        out_specs=_row_spec(cout),
        out_shape=jax.ShapeDtypeStruct((N, cout), jnp.float32),
    )


_mid2 = _mid(HID_C, HID_C)
_mid3 = _mid(HID_C, OUT_C)

_final = pl.pallas_call(
    _final_body,
    grid=_GRID,
    in_specs=[_p_spec(OUT_C), _row_spec(OUT_C), _row_spec(1), _full_spec(1, OUT_C)],
    out_specs=_row_spec(OUT_C),
    out_shape=jax.ShapeDtypeStruct((N, OUT_C), jnp.float32),
)


def kernel(x, edge_index, W1, b1, W2, b2, W3, b3):
    src = edge_index[0].astype(jnp.int32)
    dst = edge_index[1].astype(jnp.int32)
    pad = E_PAD - E
    src_p = jnp.concatenate([src, jnp.zeros((pad,), jnp.int32)])
    dst_p = jnp.concatenate([dst, jnp.full((pad,), N, jnp.int32)])

    ones16 = jnp.ones((N, 16), jnp.float32)
    degp = _agg16(ones16, src_p, dst_p)
    dinv, hs1 = _prep(degp, x, W1)
    p1 = _agg128(hs1, src_p, dst_p)
    hs2 = _mid2(p1, hs1, dinv, b1.reshape(1, -1), W2)
    p2 = _agg128(hs2, src_p, dst_p)
    hs3 = _mid3(p2, hs2, dinv, b2.reshape(1, -1), W3)
    p3 = _agg64(hs3, src_p, dst_p)
    return _final(p3, hs3, dinv, b3.reshape(1, -1))


# speedup vs baseline: 9.2724x; 1.3205x over previous
import functools

import jax
import jax.numpy as jnp
from jax import lax
from jax.experimental import pallas as pl
from jax.experimental.pallas import tpu as pltpu
from jax.experimental.pallas import tpu_sc as plsc

N = 10000
E = 320000
IN_C = 128
HID_C = 128
OUT_C = 64

NC, NS = 2, 16
NT = NC * NS
EB = 128
EPT = 10240
E_PAD = NT * EPT
NB = EPT // EB
N_PAD = 10240
RPT = N_PAD // NS
ZCH = 64

_MESH = plsc.VectorSubcoreMesh(
    core_axis_name="c", subcore_axis_name="s", num_cores=NC, num_subcores=NS
)


NIB = 4


def _make_agg(C):
    @functools.partial(
        pl.kernel,
        out_type=jax.ShapeDtypeStruct((NC, N_PAD, C), jnp.float32),
        mesh=_MESH,
        scratch_types=[
            [pltpu.VMEM((EB,), jnp.int32) for _ in range(NIB)],
            [pltpu.VMEM((EB,), jnp.int32) for _ in range(NIB)],
            pltpu.VMEM((2, EB, C), jnp.float32),
            pltpu.VMEM((ZCH, C), jnp.float32),
            pltpu.VMEM_SHARED((N_PAD, C), jnp.float32),
            pltpu.SemaphoreType.DMA,
            pltpu.SemaphoreType.DMA,
            pltpu.SemaphoreType.DMA,
        ],
        compiler_params=pltpu.CompilerParams(use_tc_tiling_on_sc=False),
    )
    def agg(hs, src, dst, out, srcb, dstb, rows, chunk, acc, sem_i, sem_g, sem_s):
        cid = lax.axis_index("c")
        sid = lax.axis_index("s")
        e0 = (cid * NS + sid) * EPT

        def si(j, jj):
            pltpu.async_copy(src.at[pl.ds(e0 + j * EB, EB)], srcb[jj], sem_i)
            pltpu.async_copy(dst.at[pl.ds(e0 + j * EB, EB)], dstb[jj], sem_i)

        def wi():
            pltpu.make_async_copy(src.at[pl.ds(0, EB)], srcb[0], sem_i).wait()
            pltpu.make_async_copy(dst.at[pl.ds(0, EB)], dstb[0], sem_i).wait()

        def sg(jj, bi):
            pltpu.async_copy(hs.at[srcb[jj]], rows.at[bi], sem_g)

        def wg():
            pltpu.make_async_copy(hs.at[srcb[0]], rows.at[0], sem_g).wait()

        def ss(jj, bi):
            pltpu.async_copy(rows.at[bi], acc.at[dstb[jj]], sem_s, add=True)

        def ws():
            pltpu.make_async_copy(rows.at[0], acc.at[dstb[0]], sem_s).wait()

        def zlane(t, _):
            chunk[t // (C // 16), pl.ds((t % (C // 16)) * 16, 16)] = jnp.zeros(
                (16,), jnp.float32
            )
            return _

        lax.fori_loop(0, ZCH * (C // 16), zlane, None)
        r0 = sid * RPT
        for k in range(RPT // ZCH):
            pltpu.async_copy(chunk, acc.at[pl.ds(r0 + k * ZCH, ZCH)], sem_s)
        for k in range(RPT // ZCH):
            pltpu.make_async_copy(chunk, acc.at[pl.ds(r0, ZCH)], sem_s).wait()
        plsc.subcore_barrier()

        si(0, 0)
        si(1, 1)
        si(2, 2)
        wi()
        sg(0, 0)

        def quad(g, _):
            j0 = g * 4
            for u in range(4):
                j = j0 + u
                jj = u % NIB
                bi = u % 2
                wg()

                @pl.when(j >= 1)
                def _():
                    ws()

                @pl.when(j + 3 < NB)
                def _():
                    si(j + 3, (u + 3) % NIB)

                @pl.when(j + 1 < NB)
                def _():
                    wi()
                    sg((u + 1) % NIB, 1 - bi)

                ss(jj, bi)
            return _

        lax.fori_loop(0, NB // 4, quad, None)
        ws()
        plsc.subcore_barrier()

        pltpu.sync_copy(acc.at[pl.ds(r0, RPT)], out.at[cid].at[pl.ds(r0, RPT)])

    return agg


def _make_deg():
    C = 16

    @functools.partial(
        pl.kernel,
        out_type=jax.ShapeDtypeStruct((NC, N_PAD, C), jnp.float32),
        mesh=_MESH,
        scratch_types=[
            [pltpu.VMEM((EB,), jnp.int32) for _ in range(NIB)],
            pltpu.VMEM((EB, C), jnp.float32),
            pltpu.VMEM((ZCH, C), jnp.float32),
            pltpu.VMEM_SHARED((N_PAD, C), jnp.float32),
            pltpu.SemaphoreType.DMA,
            pltpu.SemaphoreType.DMA,
        ],
        compiler_params=pltpu.CompilerParams(use_tc_tiling_on_sc=False),
    )
    def deg(dst, out, dstb, ones, chunk, acc, sem_i, sem_s):
        cid = lax.axis_index("c")
        sid = lax.axis_index("s")
        e0 = (cid * NS + sid) * EPT

        def si(j, jj):
            pltpu.async_copy(dst.at[pl.ds(e0 + j * EB, EB)], dstb[jj], sem_i)

        def wi():
            pltpu.make_async_copy(dst.at[pl.ds(0, EB)], dstb[0], sem_i).wait()

        def ss(jj):
            pltpu.async_copy(ones, acc.at[dstb[jj]], sem_s, add=True)

        def ws():
            pltpu.make_async_copy(ones, acc.at[dstb[0]], sem_s).wait()

        def fill(t, _):
            chunk[t // 1, pl.ds(0, 16)] = jnp.zeros((16,), jnp.float32)
            return _

        lax.fori_loop(0, ZCH, fill, None)

        def fill1(t, _):
            ones[t, pl.ds(0, 16)] = jnp.ones((16,), jnp.float32)
            return _

        lax.fori_loop(0, EB, fill1, None)
        r0 = sid * RPT
        for k in range(RPT // ZCH):
            pltpu.async_copy(chunk, acc.at[pl.ds(r0 + k * ZCH, ZCH)], sem_s)
        for k in range(RPT // ZCH):
            pltpu.make_async_copy(chunk, acc.at[pl.ds(r0, ZCH)], sem_s).wait()
        plsc.subcore_barrier()

        si(0, 0)
        si(1, 1)

        def quad(g, _):
            j0 = g * 4
            for u in range(4):
                j = j0 + u
                jj = u % NIB

                @pl.when(j >= 2)
                def _():
                    ws()

                @pl.when(j + 2 < NB)
                def _():
                    si(j + 2, (u + 2) % NIB)

                wi()
                ss(jj)
            return _

        lax.fori_loop(0, NB // 4, quad, None)
        ws()
        ws()
        plsc.subcore_barrier()
        pltpu.sync_copy(acc.at[pl.ds(r0, RPT)], out.at[cid].at[pl.ds(r0, RPT)])

    return deg


_deg16 = _make_deg()
_agg128 = _make_agg(HID_C)
_agg64 = _make_agg(OUT_C)

_BM = 1000
_GRID = (N // _BM,)


def _prep_body(degp_ref, x_ref, w_ref, dinv_ref, hs_ref):
    deg = degp_ref[0, :, 0] + degp_ref[1, :, 0] + 1.0
    dv = lax.rsqrt(deg)[:, None]
    dinv_ref[...] = dv
    hs_ref[...] = jnp.dot(x_ref[...], w_ref[...], preferred_element_type=jnp.float32) * dv


def _mid_body(p_ref, hs_ref, dinv_ref, b_ref, w_ref, o_ref):
    dv = dinv_ref[...]
    t = (p_ref[0] + p_ref[1] + hs_ref[...]) * dv + b_ref[...]
    a = jnp.maximum(t, 0.0)
    o_ref[...] = jnp.dot(a, w_ref[...], preferred_element_type=jnp.float32) * dv


def _final_body(p_ref, hs_ref, dinv_ref, b_ref, o_ref):
    t = (p_ref[0] + p_ref[1] + hs_ref[...]) * dinv_ref[...] + b_ref[...]
    m = jnp.max(t, axis=1, keepdims=True)
    lse = jnp.log(jnp.sum(jnp.exp(t - m), axis=1, keepdims=True)) + m
    o_ref[...] = t - lse


def _row_spec(c):
    return pl.BlockSpec((_BM, c), lambda i: (i, 0))


def _p_spec(c):
    return pl.BlockSpec((NC, _BM, c), lambda i: (0, i, 0))


def _full_spec(a, b):
    return pl.BlockSpec((a, b), lambda i: (0, 0))


_prep = pl.pallas_call(
    _prep_body,
    grid=_GRID,
    in_specs=[_p_spec(16), _row_spec(IN_C), _full_spec(IN_C, HID_C)],
    out_specs=[_row_spec(1), _row_spec(HID_C)],
    out_shape=[
        jax.ShapeDtypeStruct((N, 1), jnp.float32),
        jax.ShapeDtypeStruct((N, HID_C), jnp.float32),
    ],
)

def _mid(cin, cout):
    return pl.pallas_call(
        _mid_body,
        grid=_GRID,
        in_specs=[
            _p_spec(cin),
            _row_spec(cin),
            _row_spec(1),
            _full_spec(1, cin),
            _full_spec(cin, cout),
        ],
        out_specs=_row_spec(cout),
        out_shape=jax.ShapeDtypeStruct((N, cout), jnp.float32),
    )


_mid2 = _mid(HID_C, HID_C)
_mid3 = _mid(HID_C, OUT_C)

_final = pl.pallas_call(
    _final_body,
    grid=_GRID,
    in_specs=[_p_spec(OUT_C), _row_spec(OUT_C), _row_spec(1), _full_spec(1, OUT_C)],
    out_specs=_row_spec(OUT_C),
    out_shape=jax.ShapeDtypeStruct((N, OUT_C), jnp.float32),
)


def kernel(x, edge_index, W1, b1, W2, b2, W3, b3):
    src = edge_index[0].astype(jnp.int32)
    dst = edge_index[1].astype(jnp.int32)
    pad = E_PAD - E
    src_p = jnp.concatenate([src, jnp.zeros((pad,), jnp.int32)])
    dst_p = jnp.concatenate([dst, jnp.full((pad,), N, jnp.int32)])

    degp = _deg16(dst_p)
    dinv, hs1 = _prep(degp, x, W1)
    p1 = _agg128(hs1, src_p, dst_p)
    hs2 = _mid2(p1, hs1, dinv, b1.reshape(1, -1), W2)
    p2 = _agg128(hs2, src_p, dst_p)
    hs3 = _mid3(p2, hs2, dinv, b2.reshape(1, -1), W3)
    p3 = _agg64(hs3, src_p, dst_p)
    return _final(p3, hs3, dinv, b3.reshape(1, -1))


# speedup vs baseline: 21.0047x; 2.2653x over previous
import functools

import jax
import jax.numpy as jnp
from jax import lax
from jax.experimental import pallas as pl
from jax.experimental.pallas import tpu as pltpu
from jax.experimental.pallas import tpu_sc as plsc

N = 10000
E = 320000
IN_C = 128
HID_C = 128
OUT_C = 64
HC = 64

NC, NS = 2, 16
NT = NC * NS
EB = 128
EPT = 10240
E_PAD = NT * EPT
NB = EPT // EB
N_PAD = 10240
RPT = N_PAD // NS
SPT = N // NS
ZCH = 64
NIB = 4

_MESH = plsc.VectorSubcoreMesh(
    core_axis_name="c", subcore_axis_name="s", num_cores=NC, num_subcores=NS
)


def _make_agg(H):
    @functools.partial(
        pl.kernel,
        out_type=jax.ShapeDtypeStruct((NC, H, N_PAD, HC), jnp.float32),
        mesh=_MESH,
        scratch_types=[
            [pltpu.VMEM((EB,), jnp.int32) for _ in range(NIB)],
            [pltpu.VMEM((EB,), jnp.int32) for _ in range(NIB)],
            pltpu.VMEM((2, EB, HC), jnp.float32),
            pltpu.VMEM((ZCH, HC), jnp.float32),
            pltpu.VMEM_SHARED((N, HC), jnp.float32),
            pltpu.VMEM_SHARED((N_PAD, HC), jnp.float32),
            pltpu.SemaphoreType.DMA,
            pltpu.SemaphoreType.DMA,
            pltpu.SemaphoreType.DMA,
        ],
        compiler_params=pltpu.CompilerParams(use_tc_tiling_on_sc=False),
    )
    def agg(hs, src, dst, out, srcb, dstb, rows, chunk, hsp, acc, sem_i, sem_g, sem_s):
        cid = lax.axis_index("c")
        sid = lax.axis_index("s")
        e0 = (cid * NS + sid) * EPT
        r0 = sid * RPT
        s0 = sid * SPT

        def si(j, jj):
            pltpu.async_copy(src.at[pl.ds(e0 + j * EB, EB)], srcb[jj], sem_i)
            pltpu.async_copy(dst.at[pl.ds(e0 + j * EB, EB)], dstb[jj], sem_i)

        def wi():
            pltpu.make_async_copy(src.at[pl.ds(0, EB)], srcb[0], sem_i).wait()
            pltpu.make_async_copy(dst.at[pl.ds(0, EB)], dstb[0], sem_i).wait()

        def sg(jj, bi):
            pltpu.async_copy(hsp.at[srcb[jj]], rows.at[bi], sem_g)

        def wg():
            pltpu.make_async_copy(hsp.at[srcb[0]], rows.at[0], sem_g).wait()

        def ss(jj, bi):
            pltpu.async_copy(rows.at[bi], acc.at[dstb[jj]], sem_s, add=True)

        def ws():
            pltpu.make_async_copy(rows.at[0], acc.at[dstb[0]], sem_s).wait()

        def zlane(t, _):
            chunk[t // (HC // 16), pl.ds((t % (HC // 16)) * 16, 16)] = jnp.zeros(
                (16,), jnp.float32
            )
            return _

        lax.fori_loop(0, ZCH * (HC // 16), zlane, None)

        for h in range(H):
            pltpu.async_copy(hs.at[h].at[pl.ds(s0, SPT)], hsp.at[pl.ds(s0, SPT)], sem_g)
            for k in range(RPT // ZCH):
                pltpu.async_copy(chunk, acc.at[pl.ds(r0 + k * ZCH, ZCH)], sem_s)
            pltpu.make_async_copy(
                hs.at[h].at[pl.ds(s0, SPT)], hsp.at[pl.ds(s0, SPT)], sem_g
            ).wait()
            for k in range(RPT // ZCH):
                pltpu.make_async_copy(chunk, acc.at[pl.ds(r0, ZCH)], sem_s).wait()
            plsc.subcore_barrier()

            si(0, 0)
            si(1, 1)
            si(2, 2)
            wi()
            sg(0, 0)

            def quad(g, _):
                j0 = g * 4
                for u in range(4):
                    j = j0 + u
                    jj = u % NIB
                    bi = u % 2
                    wg()

                    @pl.when(j >= 1)
                    def _():
                        ws()

                    @pl.when(j + 3 < NB)
                    def _():
                        si(j + 3, (u + 3) % NIB)

                    @pl.when(j + 1 < NB)
                    def _():
                        wi()
                        sg((u + 1) % NIB, 1 - bi)

                    ss(jj, bi)
                return _

            lax.fori_loop(0, NB // 4, quad, None)
            ws()
            plsc.subcore_barrier()

            pltpu.sync_copy(acc.at[pl.ds(r0, RPT)], out.at[cid].at[h].at[pl.ds(r0, RPT)])
            if h + 1 < H:
                plsc.subcore_barrier()

    return agg


def _make_deg():
    C = 16

    @functools.partial(
        pl.kernel,
        out_type=jax.ShapeDtypeStruct((NC, N_PAD, C), jnp.float32),
        mesh=_MESH,
        scratch_types=[
            [pltpu.VMEM((EB,), jnp.int32) for _ in range(NIB)],
            pltpu.VMEM((EB, C), jnp.float32),
            pltpu.VMEM((ZCH, C), jnp.float32),
            pltpu.VMEM_SHARED((N_PAD, C), jnp.float32),
            pltpu.SemaphoreType.DMA,
            pltpu.SemaphoreType.DMA,
        ],
        compiler_params=pltpu.CompilerParams(use_tc_tiling_on_sc=False),
    )
    def deg(dst, out, dstb, ones, chunk, acc, sem_i, sem_s):
        cid = lax.axis_index("c")
        sid = lax.axis_index("s")
        e0 = (cid * NS + sid) * EPT

        def si(j, jj):
            pltpu.async_copy(dst.at[pl.ds(e0 + j * EB, EB)], dstb[jj], sem_i)

        def wi():
            pltpu.make_async_copy(dst.at[pl.ds(0, EB)], dstb[0], sem_i).wait()

        def ss(jj):
            pltpu.async_copy(ones, acc.at[dstb[jj]], sem_s, add=True)

        def ws():
            pltpu.make_async_copy(ones, acc.at[dstb[0]], sem_s).wait()

        def fill(t, _):
            chunk[t, pl.ds(0, 16)] = jnp.zeros((16,), jnp.float32)
            return _

        lax.fori_loop(0, ZCH, fill, None)

        def fill1(t, _):
            ones[t, pl.ds(0, 16)] = jnp.ones((16,), jnp.float32)
            return _

        lax.fori_loop(0, EB, fill1, None)
        r0 = sid * RPT
        for k in range(RPT // ZCH):
            pltpu.async_copy(chunk, acc.at[pl.ds(r0 + k * ZCH, ZCH)], sem_s)
        for k in range(RPT // ZCH):
            pltpu.make_async_copy(chunk, acc.at[pl.ds(r0, ZCH)], sem_s).wait()
        plsc.subcore_barrier()

        si(0, 0)
        si(1, 1)

        def quad(g, _):
            j0 = g * 4
            for u in range(4):
                j = j0 + u
                jj = u % NIB

                @pl.when(j >= 2)
                def _():
                    ws()

                @pl.when(j + 2 < NB)
                def _():
                    si(j + 2, (u + 2) % NIB)

                wi()
                ss(jj)
            return _

        lax.fori_loop(0, NB // 4, quad, None)
        ws()
        ws()
        plsc.subcore_barrier()
        pltpu.sync_copy(acc.at[pl.ds(r0, RPT)], out.at[cid].at[pl.ds(r0, RPT)])

    return deg


_deg16 = _make_deg()
_agg2 = _make_agg(2)
_agg1 = _make_agg(1)

_BM = 1000
_GRID = (N // _BM,)


def _prep_body(degp_ref, x_ref, w_ref, dinv_ref, hs_ref):
    deg = degp_ref[0, :, 0] + degp_ref[1, :, 0] + 1.0
    dv = lax.rsqrt(deg)[:, None]
    dinv_ref[...] = dv
    res = jnp.dot(x_ref[...], w_ref[...], preferred_element_type=jnp.float32) * dv
    hs_ref[0] = res[:, :HC]
    hs_ref[1] = res[:, HC:]


def _mid_body(hout, p_ref, hs_ref, dinv_ref, b_ref, w_ref, o_ref):
    hin = hs_ref.shape[0]
    dv = dinv_ref[...]
    hs = jnp.concatenate([hs_ref[h] for h in range(hin)], axis=1)
    ps = jnp.concatenate(
        [p_ref[0, h] + p_ref[1, h] for h in range(hin)], axis=1
    )
    t = (ps + hs) * dv + b_ref[...]
    a = jnp.maximum(t, 0.0)
    res = jnp.dot(a, w_ref[...], preferred_element_type=jnp.float32) * dv
    for h in range(hout):
        o_ref[h] = res[:, h * HC:(h + 1) * HC]


def _final_body(p_ref, hs_ref, dinv_ref, b_ref, o_ref):
    t = (p_ref[0, 0] + p_ref[1, 0] + hs_ref[0]) * dinv_ref[...] + b_ref[...]
    m = jnp.max(t, axis=1, keepdims=True)
    lse = jnp.log(jnp.sum(jnp.exp(t - m), axis=1, keepdims=True)) + m
    o_ref[...] = t - lse


def _row_spec(c):
    return pl.BlockSpec((_BM, c), lambda i: (i, 0))


def _h_spec(hh):
    return pl.BlockSpec((hh, _BM, HC), lambda i: (0, i, 0))


def _p_spec(hh):
    return pl.BlockSpec((NC, hh, _BM, HC), lambda i: (0, 0, i, 0))


def _full_spec(a, b):
    return pl.BlockSpec((a, b), lambda i: (0, 0))


_prep = pl.pallas_call(
    _prep_body,
    grid=_GRID,
    in_specs=[
        pl.BlockSpec((NC, _BM, 16), lambda i: (0, i, 0)),
        _row_spec(IN_C),
        _full_spec(IN_C, HID_C),
    ],
    out_specs=[_row_spec(1), _h_spec(2)],
    out_shape=[
        jax.ShapeDtypeStruct((N, 1), jnp.float32),
        jax.ShapeDtypeStruct((2, N, HC), jnp.float32),
    ],
)


def _mid(hin, hout, cin, cout):
    return pl.pallas_call(
        functools.partial(_mid_body, hout),
        grid=_GRID,
        in_specs=[
            _p_spec(hin),
            _h_spec(hin),
            _row_spec(1),
            _full_spec(1, cin),
            _full_spec(cin, cout),
        ],
        out_specs=_h_spec(hout),
        out_shape=jax.ShapeDtypeStruct((hout, N, HC), jnp.float32),
    )


_mid2 = _mid(2, 2, HID_C, HID_C)
_mid3 = _mid(2, 1, HID_C, OUT_C)

_final = pl.pallas_call(
    _final_body,
    grid=_GRID,
    in_specs=[_p_spec(1), _h_spec(1), _row_spec(1), _full_spec(1, OUT_C)],
    out_specs=_row_spec(OUT_C),
    out_shape=jax.ShapeDtypeStruct((N, OUT_C), jnp.float32),
)


def kernel(x, edge_index, W1, b1, W2, b2, W3, b3):
    src = edge_index[0].astype(jnp.int32)
    dst = edge_index[1].astype(jnp.int32)
    pad = E_PAD - E
    src_p = jnp.concatenate([src, jnp.zeros((pad,), jnp.int32)])
    dst_p = jnp.concatenate([dst, jnp.full((pad,), N, jnp.int32)])

    degp = _deg16(dst_p)
    dinv, hs1 = _prep(degp, x, W1)
    p1 = _agg2(hs1, src_p, dst_p)
    hs2 = _mid2(p1, hs1, dinv, b1.reshape(1, -1), W2)
    p2 = _agg2(hs2, src_p, dst_p)
    hs3 = _mid3(p2, hs2, dinv, b2.reshape(1, -1), W3)
    p3 = _agg1(hs3, src_p, dst_p)
    return _final(p3, hs3, dinv, b3.reshape(1, -1))


# speedup vs baseline: 24.8710x; 1.1841x over previous
import functools

import jax
import jax.numpy as jnp
from jax import lax
from jax.experimental import pallas as pl
from jax.experimental.pallas import tpu as pltpu
from jax.experimental.pallas import tpu_sc as plsc

N = 10000
E = 320000
IN_C = 128
HID_C = 128
OUT_C = 64
HC = 64

NC, NS = 2, 16
NT = NC * NS
EB = 128
EPT = E // NT
NBF = EPT // EB
TB = EPT - NBF * EB
N_PAD = 10240
RPT = N_PAD // NS
SPT = N // NS
ZCH = 64
NIB = 4

_MESH = plsc.VectorSubcoreMesh(
    core_axis_name="c", subcore_axis_name="s", num_cores=NC, num_subcores=NS
)


def _make_agg(H):
    @functools.partial(
        pl.kernel,
        out_type=jax.ShapeDtypeStruct((NC, N_PAD, 128), jnp.float32),
        mesh=_MESH,
        scratch_types=[
            [pltpu.VMEM((EB,), jnp.int32) for _ in range(NIB)],
            [pltpu.VMEM((EB,), jnp.int32) for _ in range(NIB)],
            pltpu.VMEM((TB,), jnp.int32),
            pltpu.VMEM((TB,), jnp.int32),
            pltpu.VMEM((2, EB, HC), jnp.float32),
            pltpu.VMEM((TB, HC), jnp.float32),
            pltpu.VMEM((ZCH, HC), jnp.float32),
            pltpu.VMEM_SHARED((N, HC), jnp.float32),
            pltpu.VMEM_SHARED((N_PAD, HC), jnp.float32),
            pltpu.SemaphoreType.DMA,
            pltpu.SemaphoreType.DMA,
            pltpu.SemaphoreType.DMA,
        ],
        compiler_params=pltpu.CompilerParams(use_tc_tiling_on_sc=False),
    )
    def agg(hs, edge, out, srcb, dstb, srct, dstt, rows, rowst, chunk, hsp, acc,
            sem_i, sem_g, sem_s):
        cid = lax.axis_index("c")
        sid = lax.axis_index("s")
        e0 = (cid * NS + sid) * EPT
        r0 = sid * RPT
        s0 = sid * SPT

        def si(j, jj):
            pltpu.async_copy(edge.at[0].at[pl.ds(e0 + j * EB, EB)], srcb[jj], sem_i)
            pltpu.async_copy(edge.at[1].at[pl.ds(e0 + j * EB, EB)], dstb[jj], sem_i)

        def wi():
            pltpu.make_async_copy(edge.at[0].at[pl.ds(0, EB)], srcb[0], sem_i).wait()
            pltpu.make_async_copy(edge.at[1].at[pl.ds(0, EB)], dstb[0], sem_i).wait()

        def sg(jj, bi):
            pltpu.async_copy(hsp.at[srcb[jj]], rows.at[bi], sem_g)

        def wg():
            pltpu.make_async_copy(hsp.at[srcb[0]], rows.at[0], sem_g).wait()

        def ss(jj, bi):
            pltpu.async_copy(rows.at[bi], acc.at[dstb[jj]], sem_s, add=True)

        def ws():
            pltpu.make_async_copy(rows.at[0], acc.at[dstb[0]], sem_s).wait()

        def zlane(t, _):
            chunk[t // (HC // 16), pl.ds((t % (HC // 16)) * 16, 16)] = jnp.zeros(
                (16,), jnp.float32
            )
            return _

        lax.fori_loop(0, ZCH * (HC // 16), zlane, None)

        for h in range(H):
            pltpu.async_copy(
                hs.at[pl.ds(s0, SPT), pl.ds(h * HC, HC)], hsp.at[pl.ds(s0, SPT)],
                sem_g,
            )
            for k in range(RPT // ZCH):
                pltpu.async_copy(chunk, acc.at[pl.ds(r0 + k * ZCH, ZCH)], sem_s)
            pltpu.make_async_copy(
                hs.at[pl.ds(s0, SPT), pl.ds(h * HC, HC)], hsp.at[pl.ds(s0, SPT)],
                sem_g,
            ).wait()
            for k in range(RPT // ZCH):
                pltpu.make_async_copy(chunk, acc.at[pl.ds(r0, ZCH)], sem_s).wait()
            plsc.subcore_barrier()

            si(0, 0)
            si(1, 1)
            si(2, 2)
            wi()
            sg(0, 0)

            def quad(g, _):
                j0 = g * 4
                for u in range(4):
                    j = j0 + u
                    wg()

                    @pl.when(j >= 1)
                    def _():
                        ws()

                    @pl.when(j + 3 < NBF)
                    def _():
                        si(j + 3, (u + 3) % NIB)

                    wi()
                    sg((u + 1) % NIB, 1 - (u % 2))
                    ss(u % NIB, u % 2)
                return _

            lax.fori_loop(0, NBF // 4, quad, None)
            wg()
            ws()
            wi()
            sg(1, 1)
            ss(0, 0)
            wg()
            ws()
            ss(1, 1)
            ws()
            pltpu.sync_copy(edge.at[0].at[pl.ds(e0 + NBF * EB, TB)], srct)
            pltpu.sync_copy(edge.at[1].at[pl.ds(e0 + NBF * EB, TB)], dstt)
            pltpu.async_copy(hsp.at[srct], rowst, sem_g).wait()
            pltpu.sync_copy(rowst, acc.at[dstt], add=True)
            plsc.subcore_barrier()

            pltpu.sync_copy(
                acc.at[pl.ds(r0, RPT)],
                out.at[cid].at[pl.ds(r0, RPT), pl.ds(h * HC, HC)],
            )
            if h + 1 < H:
                plsc.subcore_barrier()

    return agg


def _make_deg():
    C = 16

    @functools.partial(
        pl.kernel,
        out_type=jax.ShapeDtypeStruct((NC, N_PAD, C), jnp.float32),
        mesh=_MESH,
        scratch_types=[
            [pltpu.VMEM((EB,), jnp.int32) for _ in range(NIB)],
            pltpu.VMEM((TB,), jnp.int32),
            pltpu.VMEM((EB, C), jnp.float32),
            pltpu.VMEM((TB, C), jnp.float32),
            pltpu.VMEM((ZCH, C), jnp.float32),
            pltpu.VMEM_SHARED((N_PAD, C), jnp.float32),
            pltpu.SemaphoreType.DMA,
            pltpu.SemaphoreType.DMA,
        ],
        compiler_params=pltpu.CompilerParams(use_tc_tiling_on_sc=False),
    )
    def deg(edge, out, dstb, dstt, ones, onest, chunk, acc, sem_i, sem_s):
        cid = lax.axis_index("c")
        sid = lax.axis_index("s")
        e0 = (cid * NS + sid) * EPT

        def si(j, jj):
            pltpu.async_copy(edge.at[1].at[pl.ds(e0 + j * EB, EB)], dstb[jj], sem_i)

        def wi():
            pltpu.make_async_copy(edge.at[1].at[pl.ds(0, EB)], dstb[0], sem_i).wait()

        def ss(jj):
            pltpu.async_copy(ones, acc.at[dstb[jj]], sem_s, add=True)

        def ws():
            pltpu.make_async_copy(ones, acc.at[dstb[0]], sem_s).wait()

        def fill(t, _):
            chunk[t, pl.ds(0, 16)] = jnp.zeros((16,), jnp.float32)
            return _

        lax.fori_loop(0, ZCH, fill, None)

        def fill1(t, _):
            ones[t, pl.ds(0, 16)] = jnp.ones((16,), jnp.float32)
            return _

        lax.fori_loop(0, EB, fill1, None)

        def fill2(t, _):
            onest[t, pl.ds(0, 16)] = jnp.ones((16,), jnp.float32)
            return _

        lax.fori_loop(0, TB, fill2, None)
        r0 = sid * RPT
        for k in range(RPT // ZCH):
            pltpu.async_copy(chunk, acc.at[pl.ds(r0 + k * ZCH, ZCH)], sem_s)
        for k in range(RPT // ZCH):
            pltpu.make_async_copy(chunk, acc.at[pl.ds(r0, ZCH)], sem_s).wait()
        plsc.subcore_barrier()

        si(0, 0)
        si(1, 1)

        def quad(g, _):
            j0 = g * 4
            for u in range(4):
                j = j0 + u

                @pl.when(j >= 2)
                def _():
                    ws()

                si(j + 2, (u + 2) % NIB)
                wi()
                ss(u % NIB)
            return _

        lax.fori_loop(0, NBF // 4, quad, None)
        ws()
        wi()
        ss(0)
        ws()
        wi()
        ss(1)
        ws()
        ws()
        pltpu.sync_copy(edge.at[1].at[pl.ds(e0 + NBF * EB, TB)], dstt)
        pltpu.sync_copy(onest, acc.at[dstt], add=True)
        plsc.subcore_barrier()
        pltpu.sync_copy(acc.at[pl.ds(r0, RPT)], out.at[cid].at[pl.ds(r0, RPT)])

    return deg


_deg16 = _make_deg()
_agg2 = _make_agg(2)
_agg1 = _make_agg(1)

_BM = 1000
_GRID = (N // _BM,)


def _prep_body(degp_ref, x_ref, w_ref, dinv_ref, hs_ref):
    deg = degp_ref[0, :, 0] + degp_ref[1, :, 0] + 1.0
    dv = lax.rsqrt(deg)[:, None]
    dinv_ref[...] = dv
    hs_ref[...] = jnp.dot(x_ref[...], w_ref[...], preferred_element_type=jnp.float32) * dv


def _mid_body(p_ref, hs_ref, dinv_ref, b_ref, w_ref, o_ref):
    dv = dinv_ref[...]
    t = (p_ref[0] + p_ref[1] + hs_ref[...]) * dv + b_ref[...]
    a = jnp.maximum(t, 0.0)
    res = jnp.dot(a, w_ref[...], preferred_element_type=jnp.float32)
    if res.shape[1] == 128:
        o_ref[...] = res * dv
    else:
        o_ref[:, :OUT_C] = res * dv
        o_ref[:, OUT_C:] = jnp.zeros_like(res)


def _final_body(p_ref, hs_ref, dinv_ref, b_ref, o_ref):
    t = (
        (p_ref[0, :, :OUT_C] + p_ref[1, :, :OUT_C] + hs_ref[:, :OUT_C])
        * dinv_ref[...]
        + b_ref[...]
    )
    m = jnp.max(t, axis=1, keepdims=True)
    lse = jnp.log(jnp.sum(jnp.exp(t - m), axis=1, keepdims=True)) + m
    o_ref[...] = t - lse


def _row_spec(c):
    return pl.BlockSpec((_BM, c), lambda i: (i, 0))


def _p_spec(c):
    return pl.BlockSpec((NC, _BM, c), lambda i: (0, i, 0))


def _full_spec(a, b):
    return pl.BlockSpec((a, b), lambda i: (0, 0))


_prep = pl.pallas_call(
    _prep_body,
    grid=_GRID,
    in_specs=[_p_spec(16), _row_spec(IN_C), _full_spec(IN_C, HID_C)],
    out_specs=[_row_spec(1), _row_spec(HID_C)],
    out_shape=[
        jax.ShapeDtypeStruct((N, 1), jnp.float32),
        jax.ShapeDtypeStruct((N, HID_C), jnp.float32),
    ],
)


def _mid(cout):
    return pl.pallas_call(
        _mid_body,
        grid=_GRID,
        in_specs=[
            _p_spec(128),
            _row_spec(128),
            _row_spec(1),
            _full_spec(1, 128),
            _full_spec(128, cout),
        ],
        out_specs=_row_spec(128),
        out_shape=jax.ShapeDtypeStruct((N, 128), jnp.float32),
    )


_mid2 = _mid(HID_C)
_mid3 = _mid(OUT_C)

_final = pl.pallas_call(
    _final_body,
    grid=_GRID,
    in_specs=[_p_spec(128), _row_spec(128), _row_spec(1), _full_spec(1, OUT_C)],
    out_specs=_row_spec(OUT_C),
    out_shape=jax.ShapeDtypeStruct((N, OUT_C), jnp.float32),
)


def kernel(x, edge_index, W1, b1, W2, b2, W3, b3):
    edge = edge_index.astype(jnp.int32)
    degp = _deg16(edge)
    dinv, hs1 = _prep(degp, x, W1)
    p1 = _agg2(hs1, edge)
    hs2 = _mid2(p1, hs1, dinv, b1.reshape(1, -1), W2)
    p2 = _agg2(hs2, edge)
    hs3 = _mid3(p2, hs2, dinv, b2.reshape(1, -1), W3)
    p3 = _agg1(hs3, edge)
    return _final(p3, hs3, dinv, b3.reshape(1, -1))


# speedup vs baseline: 29.2027x; 1.1742x over previous
import functools

import jax
import jax.numpy as jnp
from jax import lax
from jax.experimental import pallas as pl
from jax.experimental.pallas import tpu as pltpu
from jax.experimental.pallas import tpu_sc as plsc

N = 10000
E = 320000
IN_C = 128
HID_C = 128
OUT_C = 64
HC = 64

NC, NS = 2, 16
NT = NC * NS
EB = 128
EPT = E // NT
NBF = EPT // EB
TB = EPT - NBF * EB
N_PAD = 10240
RPT = N_PAD // NS
SPT = N // NS
ZCH = 64
NIB = 4
NIR = 8
NRR = 4

_MESH = plsc.VectorSubcoreMesh(
    core_axis_name="c", subcore_axis_name="s", num_cores=NC, num_subcores=NS
)


def _make_agg(H):
    @functools.partial(
        pl.kernel,
        out_type=jax.ShapeDtypeStruct((NC, N_PAD, 128), jnp.float32),
        mesh=_MESH,
        scratch_types=[
            [pltpu.VMEM((EB,), jnp.int32) for _ in range(NIR)],
            [pltpu.VMEM((EB,), jnp.int32) for _ in range(NIR)],
            pltpu.VMEM((TB,), jnp.int32),
            pltpu.VMEM((TB,), jnp.int32),
            pltpu.VMEM((NRR, EB, HC), jnp.float32),
            pltpu.VMEM((TB, HC), jnp.float32),
            pltpu.VMEM((ZCH, HC), jnp.float32),
            pltpu.VMEM_SHARED((N, HC), jnp.float32),
            pltpu.VMEM_SHARED((N_PAD, HC), jnp.float32),
            pltpu.SemaphoreType.DMA,
            pltpu.SemaphoreType.DMA,
            pltpu.SemaphoreType.DMA,
        ],
        compiler_params=pltpu.CompilerParams(use_tc_tiling_on_sc=False),
    )
    def agg(hs, edge, out, srcb, dstb, srct, dstt, rows, rowst, chunk, hsp, acc,
            sem_i, sem_g, sem_s):
        cid = lax.axis_index("c")
        sid = lax.axis_index("s")
        e0 = (cid * NS + sid) * EPT
        r0 = sid * RPT
        s0 = sid * SPT

        def si(j, jj):
            pltpu.async_copy(edge.at[0].at[pl.ds(e0 + j * EB, EB)], srcb[jj], sem_i)
            pltpu.async_copy(edge.at[1].at[pl.ds(e0 + j * EB, EB)], dstb[jj], sem_i)

        def wi():
            pltpu.make_async_copy(edge.at[0].at[pl.ds(0, EB)], srcb[0], sem_i).wait()
            pltpu.make_async_copy(edge.at[1].at[pl.ds(0, EB)], dstb[0], sem_i).wait()

        def sg(jj, bi):
            pltpu.async_copy(hsp.at[srcb[jj]], rows.at[bi], sem_g)

        def wg():
            pltpu.make_async_copy(hsp.at[srcb[0]], rows.at[0], sem_g).wait()

        def ss(jj, bi):
            pltpu.async_copy(rows.at[bi], acc.at[dstb[jj]], sem_s, add=True)

        def ws():
            pltpu.make_async_copy(rows.at[0], acc.at[dstb[0]], sem_s).wait()

        def zlane(t, _):
            chunk[t // (HC // 16), pl.ds((t % (HC // 16)) * 16, 16)] = jnp.zeros(
                (16,), jnp.float32
            )
            return _

        lax.fori_loop(0, ZCH * (HC // 16), zlane, None)

        for h in range(H):
            pltpu.async_copy(
                hs.at[pl.ds(s0, SPT), pl.ds(h * HC, HC)], hsp.at[pl.ds(s0, SPT)],
                sem_g,
            )
            for k in range(RPT // ZCH):
                pltpu.async_copy(chunk, acc.at[pl.ds(r0 + k * ZCH, ZCH)], sem_s)
            pltpu.make_async_copy(
                hs.at[pl.ds(s0, SPT), pl.ds(h * HC, HC)], hsp.at[pl.ds(s0, SPT)],
                sem_g,
            ).wait()
            for k in range(RPT // ZCH):
                pltpu.make_async_copy(chunk, acc.at[pl.ds(r0, ZCH)], sem_s).wait()
            plsc.subcore_barrier()

            si(0, 0)
            si(1, 1)
            si(2, 2)
            si(3, 3)
            wi()
            sg(0, 0)
            wi()
            sg(1, 1)

            def step(j, u, static=True):
                wg()
                ss(u % NRR, u)
                if not static or j >= 2:
                    ws()
                if not static or j + 4 < NBF:
                    si(j + 4, (u + 4) % NIR)
                if not static or j + 2 < NBF:
                    wi()
                    sg((u + 2) % NRR, (u + 2) % NIR)

            for j in range(NIR):
                step(j, j)

            def oct_(g, _):
                j0 = NIR + g * NIR
                for u in range(NIR):
                    step(j0 + u, u, static=False)
                return _

            lax.fori_loop(0, (NBF - NIR) // NIR, oct_, None)
            for j in range(NBF - (NBF - NIR) % NIR, NBF):
                step(j, j % NIR)
            ws()
            ws()
            pltpu.sync_copy(edge.at[0].at[pl.ds(e0 + NBF * EB, TB)], srct)
            pltpu.sync_copy(edge.at[1].at[pl.ds(e0 + NBF * EB, TB)], dstt)
            pltpu.async_copy(hsp.at[srct], rowst, sem_g).wait()
            pltpu.sync_copy(rowst, acc.at[dstt], add=True)
            plsc.subcore_barrier()

            pltpu.sync_copy(
                acc.at[pl.ds(r0, RPT)],
                out.at[cid].at[pl.ds(r0, RPT), pl.ds(h * HC, HC)],
            )
            if h + 1 < H:
                plsc.subcore_barrier()

    return agg


def _make_deg():
    C = 16

    @functools.partial(
        pl.kernel,
        out_type=jax.ShapeDtypeStruct((NC, N_PAD, C), jnp.float32),
        mesh=_MESH,
        scratch_types=[
            [pltpu.VMEM((EB,), jnp.int32) for _ in range(NIB)],
            pltpu.VMEM((TB,), jnp.int32),
            pltpu.VMEM((EB, C), jnp.float32),
            pltpu.VMEM((TB, C), jnp.float32),
            pltpu.VMEM((ZCH, C), jnp.float32),
            pltpu.VMEM_SHARED((N_PAD, C), jnp.float32),
            pltpu.SemaphoreType.DMA,
            pltpu.SemaphoreType.DMA,
        ],
        compiler_params=pltpu.CompilerParams(use_tc_tiling_on_sc=False),
    )
    def deg(edge, out, dstb, dstt, ones, onest, chunk, acc, sem_i, sem_s):
        cid = lax.axis_index("c")
        sid = lax.axis_index("s")
        e0 = (cid * NS + sid) * EPT

        def si(j, jj):
            pltpu.async_copy(edge.at[1].at[pl.ds(e0 + j * EB, EB)], dstb[jj], sem_i)

        def wi():
            pltpu.make_async_copy(edge.at[1].at[pl.ds(0, EB)], dstb[0], sem_i).wait()

        def ss(jj):
            pltpu.async_copy(ones, acc.at[dstb[jj]], sem_s, add=True)

        def ws():
            pltpu.make_async_copy(ones, acc.at[dstb[0]], sem_s).wait()

        def fill(t, _):
            chunk[t, pl.ds(0, 16)] = jnp.zeros((16,), jnp.float32)
            return _

        lax.fori_loop(0, ZCH, fill, None)

        def fill1(t, _):
            ones[t, pl.ds(0, 16)] = jnp.ones((16,), jnp.float32)
            return _

        lax.fori_loop(0, EB, fill1, None)

        def fill2(t, _):
            onest[t, pl.ds(0, 16)] = jnp.ones((16,), jnp.float32)
            return _

        lax.fori_loop(0, TB, fill2, None)
        r0 = sid * RPT
        for k in range(RPT // ZCH):
            pltpu.async_copy(chunk, acc.at[pl.ds(r0 + k * ZCH, ZCH)], sem_s)
        for k in range(RPT // ZCH):
            pltpu.make_async_copy(chunk, acc.at[pl.ds(r0, ZCH)], sem_s).wait()
        plsc.subcore_barrier()

        si(0, 0)
        si(1, 1)

        def quad(g, _):
            j0 = g * 4
            for u in range(4):
                j = j0 + u

                @pl.when(j >= 2)
                def _():
                    ws()

                si(j + 2, (u + 2) % NIB)
                wi()
                ss(u % NIB)
            return _

        lax.fori_loop(0, NBF // 4, quad, None)
        ws()
        wi()
        ss(0)
        ws()
        wi()
        ss(1)
        ws()
        ws()
        pltpu.sync_copy(edge.at[1].at[pl.ds(e0 + NBF * EB, TB)], dstt)
        pltpu.sync_copy(onest, acc.at[dstt], add=True)
        plsc.subcore_barrier()
        pltpu.sync_copy(acc.at[pl.ds(r0, RPT)], out.at[cid].at[pl.ds(r0, RPT)])

    return deg


_deg16 = _make_deg()
_agg2 = _make_agg(2)
_agg1 = _make_agg(1)

_BM = 1000
_GRID = (N // _BM,)


def _prep_body(degp_ref, x_ref, w_ref, dinv_ref, hs_ref):
    deg = degp_ref[0, :, 0] + degp_ref[1, :, 0] + 1.0
    dv = lax.rsqrt(deg)[:, None]
    dinv_ref[...] = dv
    hs_ref[...] = jnp.dot(x_ref[...], w_ref[...], preferred_element_type=jnp.float32) * dv


def _mid_body(p_ref, hs_ref, dinv_ref, b_ref, w_ref, o_ref):
    dv = dinv_ref[...]
    t = (p_ref[0] + p_ref[1] + hs_ref[...]) * dv + b_ref[...]
    a = jnp.maximum(t, 0.0)
    res = jnp.dot(a, w_ref[...], preferred_element_type=jnp.float32)
    if res.shape[1] == 128:
        o_ref[...] = res * dv
    else:
        o_ref[:, :OUT_C] = res * dv
        o_ref[:, OUT_C:] = jnp.zeros_like(res)


def _final_body(p_ref, hs_ref, dinv_ref, b_ref, o_ref):
    t = (
        (p_ref[0, :, :OUT_C] + p_ref[1, :, :OUT_C] + hs_ref[:, :OUT_C])
        * dinv_ref[...]
        + b_ref[...]
    )
    m = jnp.max(t, axis=1, keepdims=True)
    lse = jnp.log(jnp.sum(jnp.exp(t - m), axis=1, keepdims=True)) + m
    o_ref[...] = t - lse


def _row_spec(c):
    return pl.BlockSpec((_BM, c), lambda i: (i, 0))


def _p_spec(c):
    return pl.BlockSpec((NC, _BM, c), lambda i: (0, i, 0))


def _full_spec(a, b):
    return pl.BlockSpec((a, b), lambda i: (0, 0))


_prep = pl.pallas_call(
    _prep_body,
    grid=_GRID,
    in_specs=[_p_spec(16), _row_spec(IN_C), _full_spec(IN_C, HID_C)],
    out_specs=[_row_spec(1), _row_spec(HID_C)],
    out_shape=[
        jax.ShapeDtypeStruct((N, 1), jnp.float32),
        jax.ShapeDtypeStruct((N, HID_C), jnp.float32),
    ],
)


def _mid(cout):
    return pl.pallas_call(
        _mid_body,
        grid=_GRID,
        in_specs=[
            _p_spec(128),
            _row_spec(128),
            _row_spec(1),
            _full_spec(1, 128),
            _full_spec(128, cout),
        ],
        out_specs=_row_spec(128),
        out_shape=jax.ShapeDtypeStruct((N, 128), jnp.float32),
    )


_mid2 = _mid(HID_C)
_mid3 = _mid(OUT_C)

_final = pl.pallas_call(
    _final_body,
    grid=_GRID,
    in_specs=[_p_spec(128), _row_spec(128), _row_spec(1), _full_spec(1, OUT_C)],
    out_specs=_row_spec(OUT_C),
    out_shape=jax.ShapeDtypeStruct((N, OUT_C), jnp.float32),
)


def kernel(x, edge_index, W1, b1, W2, b2, W3, b3):
    edge = edge_index.astype(jnp.int32)
    degp = _deg16(edge)
    dinv, hs1 = _prep(degp, x, W1)
    p1 = _agg2(hs1, edge)
    hs2 = _mid2(p1, hs1, dinv, b1.reshape(1, -1), W2)
    p2 = _agg2(hs2, edge)
    hs3 = _mid3(p2, hs2, dinv, b2.reshape(1, -1), W3)
    p3 = _agg1(hs3, edge)
    return _final(p3, hs3, dinv, b3.reshape(1, -1))


# speedup vs baseline: 29.2238x; 1.0007x over previous
import functools

import jax
import jax.numpy as jnp
from jax import lax
from jax.experimental import pallas as pl
from jax.experimental.pallas import tpu as pltpu
from jax.experimental.pallas import tpu_sc as plsc

N = 10000
E = 320000
IN_C = 128
HID_C = 128
OUT_C = 64
HC = 64

NC, NS = 2, 16
NT = NC * NS
EB = 128
EPT = E // NT
NBF = EPT // EB
TB = EPT - NBF * EB
N_PAD = 10240
RPT = N_PAD // NS
SPT = N // NS
ZCH = 64
NIB = 4
NIR = 8
NRR = 4

_MESH = plsc.VectorSubcoreMesh(
    core_axis_name="c", subcore_axis_name="s", num_cores=NC, num_subcores=NS
)


def _make_agg(H):
    @functools.partial(
        pl.kernel,
        out_type=jax.ShapeDtypeStruct((NC, N_PAD, 128), jnp.float32),
        mesh=_MESH,
        scratch_types=[
            [pltpu.VMEM((EB,), jnp.int32) for _ in range(NIR)],
            [pltpu.VMEM((EB,), jnp.int32) for _ in range(NIR)],
            pltpu.VMEM((TB,), jnp.int32),
            pltpu.VMEM((TB,), jnp.int32),
            pltpu.VMEM((NRR, EB, HC), jnp.float32),
            pltpu.VMEM((TB, HC), jnp.float32),
            pltpu.VMEM((ZCH, HC), jnp.float32),
            pltpu.VMEM_SHARED((N, HC), jnp.float32),
            pltpu.VMEM_SHARED((N_PAD, HC), jnp.float32),
            [pltpu.SemaphoreType.DMA for _ in range(4)],
            [pltpu.SemaphoreType.DMA for _ in range(2)],
            [pltpu.SemaphoreType.DMA for _ in range(2)],
        ],
        compiler_params=pltpu.CompilerParams(use_tc_tiling_on_sc=False),
    )
    def agg(hs, edge, out, srcb, dstb, srct, dstt, rows, rowst, chunk, hsp, acc,
            sem_i, sem_g, sem_s):
        cid = lax.axis_index("c")
        sid = lax.axis_index("s")
        e0 = (cid * NS + sid) * EPT
        r0 = sid * RPT
        s0 = sid * SPT

        def si(j, jj):
            pltpu.async_copy(edge.at[0].at[pl.ds(e0 + j * EB, EB)], srcb[jj],
                             sem_i[jj % 4])
            pltpu.async_copy(edge.at[1].at[pl.ds(e0 + j * EB, EB)], dstb[jj],
                             sem_i[jj % 4])

        def wi(jj):
            pltpu.make_async_copy(edge.at[0].at[pl.ds(0, EB)], srcb[0],
                                  sem_i[jj % 4]).wait()
            pltpu.make_async_copy(edge.at[1].at[pl.ds(0, EB)], dstb[0],
                                  sem_i[jj % 4]).wait()

        def sg(jj, bi):
            pltpu.async_copy(hsp.at[srcb[jj]], rows.at[bi], sem_g[bi % 2])

        def wg(bi):
            pltpu.make_async_copy(hsp.at[srcb[0]], rows.at[0], sem_g[bi % 2]).wait()

        def ss(jj, bi):
            pltpu.async_copy(rows.at[bi], acc.at[dstb[jj]], sem_s[bi % 2], add=True)

        def ws(bi):
            pltpu.make_async_copy(rows.at[0], acc.at[dstb[0]], sem_s[bi % 2]).wait()

        def zlane(t, _):
            chunk[t // (HC // 16), pl.ds((t % (HC // 16)) * 16, 16)] = jnp.zeros(
                (16,), jnp.float32
            )
            return _

        lax.fori_loop(0, ZCH * (HC // 16), zlane, None)

        for h in range(H):
            pltpu.async_copy(
                hs.at[pl.ds(s0, SPT), pl.ds(h * HC, HC)], hsp.at[pl.ds(s0, SPT)],
                sem_g[0],
            )
            for k in range(RPT // ZCH):
                pltpu.async_copy(chunk, acc.at[pl.ds(r0 + k * ZCH, ZCH)],
                                 sem_s[k % 2])
            pltpu.make_async_copy(
                hs.at[pl.ds(s0, SPT), pl.ds(h * HC, HC)], hsp.at[pl.ds(s0, SPT)],
                sem_g[0],
            ).wait()
            for k in range(RPT // ZCH):
                pltpu.make_async_copy(chunk, acc.at[pl.ds(r0, ZCH)],
                                      sem_s[k % 2]).wait()
            plsc.subcore_barrier()

            si(0, 0)
            si(1, 1)
            si(2, 2)
            si(3, 3)
            wi(0)
            sg(0, 0)
            wi(1)
            sg(1, 1)

            def step(j, u, static=True):
                wg(u % NRR)
                if not static or j >= 2:
                    ws(u % 2)
                ss(u, u % NRR)
                if not static or j + 4 < NBF:
                    si(j + 4, (u + 4) % NIR)
                if not static or j + 2 < NBF:
                    wi((u + 2) % NIR)
                    sg((u + 2) % NIR, (u + 2) % NRR)

            for j in range(NIR):
                step(j, j)

            def oct_(g, _):
                j0 = NIR + g * NIR
                for u in range(NIR):
                    step(j0 + u, u, static=False)
                return _

            lax.fori_loop(0, (NBF - NIR) // NIR, oct_, None)
            for j in range(NBF - (NBF - NIR) % NIR, NBF):
                step(j, j % NIR)
            ws((NBF - 2) % 2)
            ws((NBF - 1) % 2)
            pltpu.sync_copy(edge.at[0].at[pl.ds(e0 + NBF * EB, TB)], srct)
            pltpu.sync_copy(edge.at[1].at[pl.ds(e0 + NBF * EB, TB)], dstt)
            pltpu.async_copy(hsp.at[srct], rowst, sem_g[0]).wait()
            pltpu.sync_copy(rowst, acc.at[dstt], add=True)
            plsc.subcore_barrier()

            pltpu.sync_copy(
                acc.at[pl.ds(r0, RPT)],
                out.at[cid].at[pl.ds(r0, RPT), pl.ds(h * HC, HC)],
            )
            if h + 1 < H:
                plsc.subcore_barrier()

    return agg


def _make_deg():
    C = 16

    @functools.partial(
        pl.kernel,
        out_type=jax.ShapeDtypeStruct((NC, N_PAD, C), jnp.float32),
        mesh=_MESH,
        scratch_types=[
            [pltpu.VMEM((EB,), jnp.int32) for _ in range(NIB)],
            pltpu.VMEM((TB,), jnp.int32),
            pltpu.VMEM((EB, C), jnp.float32),
            pltpu.VMEM((TB, C), jnp.float32),
            pltpu.VMEM((ZCH, C), jnp.float32),
            pltpu.VMEM_SHARED((N_PAD, C), jnp.float32),
            pltpu.SemaphoreType.DMA,
            pltpu.SemaphoreType.DMA,
        ],
        compiler_params=pltpu.CompilerParams(use_tc_tiling_on_sc=False),
    )
    def deg(edge, out, dstb, dstt, ones, onest, chunk, acc, sem_i, sem_s):
        cid = lax.axis_index("c")
        sid = lax.axis_index("s")
        e0 = (cid * NS + sid) * EPT

        def si(j, jj):
            pltpu.async_copy(edge.at[1].at[pl.ds(e0 + j * EB, EB)], dstb[jj], sem_i)

        def wi():
            pltpu.make_async_copy(edge.at[1].at[pl.ds(0, EB)], dstb[0], sem_i).wait()

        def ss(jj):
            pltpu.async_copy(ones, acc.at[dstb[jj]], sem_s, add=True)

        def ws():
            pltpu.make_async_copy(ones, acc.at[dstb[0]], sem_s).wait()

        def fill(t, _):
            chunk[t, pl.ds(0, 16)] = jnp.zeros((16,), jnp.float32)
            return _

        lax.fori_loop(0, ZCH, fill, None)

        def fill1(t, _):
            ones[t, pl.ds(0, 16)] = jnp.ones((16,), jnp.float32)
            return _

        lax.fori_loop(0, EB, fill1, None)

        def fill2(t, _):
            onest[t, pl.ds(0, 16)] = jnp.ones((16,), jnp.float32)
            return _

        lax.fori_loop(0, TB, fill2, None)
        r0 = sid * RPT
        for k in range(RPT // ZCH):
            pltpu.async_copy(chunk, acc.at[pl.ds(r0 + k * ZCH, ZCH)], sem_s)
        for k in range(RPT // ZCH):
            pltpu.make_async_copy(chunk, acc.at[pl.ds(r0, ZCH)], sem_s).wait()
        plsc.subcore_barrier()

        si(0, 0)
        si(1, 1)

        def quad(g, _):
            j0 = g * 4
            for u in range(4):
                j = j0 + u

                @pl.when(j >= 2)
                def _():
                    ws()

                si(j + 2, (u + 2) % NIB)
                wi()
                ss(u % NIB)
            return _

        lax.fori_loop(0, NBF // 4, quad, None)
        ws()
        wi()
        ss(0)
        ws()
        wi()
        ss(1)
        ws()
        ws()
        pltpu.sync_copy(edge.at[1].at[pl.ds(e0 + NBF * EB, TB)], dstt)
        pltpu.sync_copy(onest, acc.at[dstt], add=True)
        plsc.subcore_barrier()
        pltpu.sync_copy(acc.at[pl.ds(r0, RPT)], out.at[cid].at[pl.ds(r0, RPT)])

    return deg


_deg16 = _make_deg()
_agg2 = _make_agg(2)
_agg1 = _make_agg(1)

_BM = 1000
_GRID = (N // _BM,)


def _prep_body(degp_ref, x_ref, w_ref, dinv_ref, hs_ref):
    deg = degp_ref[0, :, 0] + degp_ref[1, :, 0] + 1.0
    dv = lax.rsqrt(deg)[:, None]
    dinv_ref[...] = dv
    hs_ref[...] = jnp.dot(x_ref[...], w_ref[...], preferred_element_type=jnp.float32) * dv


def _mid_body(p_ref, hs_ref, dinv_ref, b_ref, w_ref, o_ref):
    dv = dinv_ref[...]
    t = (p_ref[0] + p_ref[1] + hs_ref[...]) * dv + b_ref[...]
    a = jnp.maximum(t, 0.0)
    res = jnp.dot(a, w_ref[...], preferred_element_type=jnp.float32)
    if res.shape[1] == 128:
        o_ref[...] = res * dv
    else:
        o_ref[:, :OUT_C] = res * dv
        o_ref[:, OUT_C:] = jnp.zeros_like(res)


def _final_body(p_ref, hs_ref, dinv_ref, b_ref, o_ref):
    t = (
        (p_ref[0, :, :OUT_C] + p_ref[1, :, :OUT_C] + hs_ref[:, :OUT_C])
        * dinv_ref[...]
        + b_ref[...]
    )
    m = jnp.max(t, axis=1, keepdims=True)
    lse = jnp.log(jnp.sum(jnp.exp(t - m), axis=1, keepdims=True)) + m
    o_ref[...] = t - lse


def _row_spec(c):
    return pl.BlockSpec((_BM, c), lambda i: (i, 0))


def _p_spec(c):
    return pl.BlockSpec((NC, _BM, c), lambda i: (0, i, 0))


def _full_spec(a, b):
    return pl.BlockSpec((a, b), lambda i: (0, 0))


_prep = pl.pallas_call(
    _prep_body,
    grid=_GRID,
    in_specs=[_p_spec(16), _row_spec(IN_C), _full_spec(IN_C, HID_C)],
    out_specs=[_row_spec(1), _row_spec(HID_C)],
    out_shape=[
        jax.ShapeDtypeStruct((N, 1), jnp.float32),
        jax.ShapeDtypeStruct((N, HID_C), jnp.float32),
    ],
)


def _mid(cout):
    return pl.pallas_call(
        _mid_body,
        grid=_GRID,
        in_specs=[
            _p_spec(128),
            _row_spec(128),
            _row_spec(1),
            _full_spec(1, 128),
            _full_spec(128, cout),
        ],
        out_specs=_row_spec(128),
        out_shape=jax.ShapeDtypeStruct((N, 128), jnp.float32),
    )


_mid2 = _mid(HID_C)
_mid3 = _mid(OUT_C)

_final = pl.pallas_call(
    _final_body,
    grid=_GRID,
    in_specs=[_p_spec(128), _row_spec(128), _row_spec(1), _full_spec(1, OUT_C)],
    out_specs=_row_spec(OUT_C),
    out_shape=jax.ShapeDtypeStruct((N, OUT_C), jnp.float32),
)


def kernel(x, edge_index, W1, b1, W2, b2, W3, b3):
    edge = edge_index.astype(jnp.int32)
    degp = _deg16(edge)
    dinv, hs1 = _prep(degp, x, W1)
    p1 = _agg2(hs1, edge)
    hs2 = _mid2(p1, hs1, dinv, b1.reshape(1, -1), W2)
    p2 = _agg2(hs2, edge)
    hs3 = _mid3(p2, hs2, dinv, b2.reshape(1, -1), W3)
    p3 = _agg1(hs3, edge)
    return _final(p3, hs3, dinv, b3.reshape(1, -1))


# speedup vs baseline: 30.7925x; 1.0537x over previous
import functools

import jax
import jax.numpy as jnp
from jax import lax
from jax.experimental import pallas as pl
from jax.experimental.pallas import tpu as pltpu
from jax.experimental.pallas import tpu_sc as plsc

N = 10000
E = 320000
IN_C = 128
HID_C = 128
OUT_C = 64
HC = 64

NC, NS = 2, 16
NT = NC * NS
EB = 128
EPT = E // NT
NBF = EPT // EB
TB = EPT - NBF * EB
N_PAD = 10240
RPT = N_PAD // NS
SPT = N // NS
ZCH = 64
NIB = 4
NIR = 8
NRR = 4

_MESH = plsc.VectorSubcoreMesh(
    core_axis_name="c", subcore_axis_name="s", num_cores=NC, num_subcores=NS
)


def _make_agg(H):
    @functools.partial(
        pl.kernel,
        out_type=jax.ShapeDtypeStruct((NC, N_PAD, 128), jnp.float32),
        mesh=_MESH,
        scratch_types=[
            [pltpu.VMEM((EB,), jnp.int32) for _ in range(NIR)],
            [pltpu.VMEM((EB,), jnp.int32) for _ in range(NIR)],
            pltpu.VMEM((TB,), jnp.int32),
            pltpu.VMEM((TB,), jnp.int32),
            pltpu.VMEM((NRR, EB, HC), jnp.float32),
            pltpu.VMEM((TB, HC), jnp.float32),
            pltpu.VMEM((ZCH, HC), jnp.float32),
            pltpu.VMEM_SHARED((N, HC), jnp.float32),
            pltpu.VMEM_SHARED((N_PAD, HC), jnp.float32),
            [pltpu.SemaphoreType.DMA for _ in range(4)],
            [pltpu.SemaphoreType.DMA for _ in range(2)],
            [pltpu.SemaphoreType.DMA for _ in range(2)],
        ],
        compiler_params=pltpu.CompilerParams(use_tc_tiling_on_sc=False),
    )
    def agg(hs, edge, out, srcb, dstb, srct, dstt, rows, rowst, chunk, hsp, acc,
            sem_i, sem_g, sem_s):
        cid = lax.axis_index("c")
        sid = lax.axis_index("s")
        e0 = (cid * NS + sid) * EPT
        r0 = sid * RPT
        s0 = sid * SPT

        def si(j, jj):
            pltpu.async_copy(edge.at[0].at[pl.ds(e0 + j * EB, EB)], srcb[jj],
                             sem_i[jj % 4])
            pltpu.async_copy(edge.at[1].at[pl.ds(e0 + j * EB, EB)], dstb[jj],
                             sem_i[jj % 4])

        def wi(jj):
            pltpu.make_async_copy(edge.at[0].at[pl.ds(0, EB)], srcb[0],
                                  sem_i[jj % 4]).wait()
            pltpu.make_async_copy(edge.at[1].at[pl.ds(0, EB)], dstb[0],
                                  sem_i[jj % 4]).wait()

        def sg(jj, bi):
            pltpu.async_copy(hsp.at[srcb[jj]], rows.at[bi], sem_g[bi % 2])

        def wg(bi):
            pltpu.make_async_copy(hsp.at[srcb[0]], rows.at[0], sem_g[bi % 2]).wait()

        def ss(jj, bi):
            pltpu.async_copy(rows.at[bi], acc.at[dstb[jj]], sem_s[bi % 2], add=True)

        def ws(bi):
            pltpu.make_async_copy(rows.at[0], acc.at[dstb[0]], sem_s[bi % 2]).wait()

        def zlane(t, _):
            chunk[t // (HC // 16), pl.ds((t % (HC // 16)) * 16, 16)] = jnp.zeros(
                (16,), jnp.float32
            )
            return _

        lax.fori_loop(0, ZCH * (HC // 16), zlane, None)

        for h in range(H):
            pltpu.async_copy(
                hs.at[pl.ds(s0, SPT), pl.ds(h * HC, HC)], hsp.at[pl.ds(s0, SPT)],
                sem_g[0],
            )
            for k in range(RPT // ZCH):
                pltpu.async_copy(chunk, acc.at[pl.ds(r0 + k * ZCH, ZCH)],
                                 sem_s[k % 2])
            pltpu.make_async_copy(
                hs.at[pl.ds(s0, SPT), pl.ds(h * HC, HC)], hsp.at[pl.ds(s0, SPT)],
                sem_g[0],
            ).wait()
            for k in range(RPT // ZCH):
                pltpu.make_async_copy(chunk, acc.at[pl.ds(r0, ZCH)],
                                      sem_s[k % 2]).wait()
            plsc.subcore_barrier()

            si(0, 0)
            si(1, 1)
            si(2, 2)
            si(3, 3)
            wi(0)
            sg(0, 0)
            wi(1)
            sg(1, 1)

            def step(j, u, static=True):
                wg(u % NRR)
                if not static or j >= 2:
                    ws(u % 2)
                ss(u, u % NRR)
                if not static or j + 4 < NBF:
                    si(j + 4, (u + 4) % NIR)
                if not static or j + 2 < NBF:
                    wi((u + 2) % NIR)
                    sg((u + 2) % NIR, (u + 2) % NRR)

            for j in range(NIR):
                step(j, j)

            def oct_(g, _):
                j0 = NIR + g * NIR
                for u in range(NIR):
                    step(j0 + u, u, static=False)
                return _

            lax.fori_loop(0, (NBF - NIR) // NIR, oct_, None)
            for j in range(NBF - (NBF - NIR) % NIR, NBF):
                step(j, j % NIR)
            ws((NBF - 2) % 2)
            ws((NBF - 1) % 2)
            pltpu.sync_copy(edge.at[0].at[pl.ds(e0 + NBF * EB, TB)], srct)
            pltpu.sync_copy(edge.at[1].at[pl.ds(e0 + NBF * EB, TB)], dstt)
            pltpu.async_copy(hsp.at[srct], rowst, sem_g[0]).wait()
            pltpu.sync_copy(rowst, acc.at[dstt], add=True)
            plsc.subcore_barrier()

            pltpu.sync_copy(
                acc.at[pl.ds(r0, RPT)],
                out.at[cid].at[pl.ds(r0, RPT), pl.ds(h * HC, HC)],
            )
            if h + 1 < H:
                plsc.subcore_barrier()

    return agg


def _make_deg():
    C = 16

    @functools.partial(
        pl.kernel,
        out_type=jax.ShapeDtypeStruct((NC, N_PAD, 128), jnp.float32),
        mesh=_MESH,
        scratch_types=[
            [pltpu.VMEM((EB,), jnp.int32) for _ in range(NIR)],
            pltpu.VMEM((TB,), jnp.int32),
            pltpu.VMEM((EB, C), jnp.float32),
            pltpu.VMEM((TB, C), jnp.float32),
            pltpu.VMEM((ZCH, C), jnp.float32),
            pltpu.VMEM_SHARED((N_PAD, C), jnp.float32),
            [pltpu.SemaphoreType.DMA for _ in range(4)],
            [pltpu.SemaphoreType.DMA for _ in range(4)],
        ],
        compiler_params=pltpu.CompilerParams(use_tc_tiling_on_sc=False),
    )
    def deg(edge, out, dstb, dstt, ones, onest, chunk, acc, sem_i, sem_s):
        cid = lax.axis_index("c")
        sid = lax.axis_index("s")
        e0 = (cid * NS + sid) * EPT

        def si(j, jj):
            pltpu.async_copy(edge.at[1].at[pl.ds(e0 + j * EB, EB)], dstb[jj],
                             sem_i[jj % 4])

        def wi(jj):
            pltpu.make_async_copy(edge.at[1].at[pl.ds(0, EB)], dstb[0],
                                  sem_i[jj % 4]).wait()

        def ss(jj):
            pltpu.async_copy(ones, acc.at[dstb[jj]], sem_s[jj % 4], add=True)

        def ws(jj):
            pltpu.make_async_copy(ones, acc.at[dstb[0]], sem_s[jj % 4]).wait()

        def fill(t, _):
            chunk[t, pl.ds(0, 16)] = jnp.zeros((16,), jnp.float32)
            return _

        lax.fori_loop(0, ZCH, fill, None)

        def fill1(t, _):
            ones[t, pl.ds(0, 16)] = jnp.ones((16,), jnp.float32)
            return _

        lax.fori_loop(0, EB, fill1, None)

        def fill2(t, _):
            onest[t, pl.ds(0, 16)] = jnp.ones((16,), jnp.float32)
            return _

        lax.fori_loop(0, TB, fill2, None)
        r0 = sid * RPT
        for k in range(RPT // ZCH):
            pltpu.async_copy(chunk, acc.at[pl.ds(r0 + k * ZCH, ZCH)],
                             sem_s[k % 4])
        for k in range(RPT // ZCH):
            pltpu.make_async_copy(chunk, acc.at[pl.ds(r0, ZCH)],
                                  sem_s[k % 4]).wait()
        plsc.subcore_barrier()

        si(0, 0)
        si(1, 1)
        si(2, 2)
        si(3, 3)

        def step(j, u, static=True):
            if not static or j >= 4:
                ws((u + 4) % NIR)
            wi(u)
            if not static or j + 4 < NBF:
                si(j + 4, (u + 4) % NIR)
            ss(u)

        for j in range(NIR):
            step(j, j)

        def oct_(g, _):
            j0 = NIR + g * NIR
            for u in range(NIR):
                step(j0 + u, u, static=False)
            return _

        lax.fori_loop(0, (NBF - NIR) // NIR, oct_, None)
        for j in range(NBF - (NBF - NIR) % NIR, NBF):
            step(j, j % NIR)
        for j in range(NBF - 4, NBF):
            ws(j % NIR)
        pltpu.sync_copy(edge.at[1].at[pl.ds(e0 + NBF * EB, TB)], dstt)
        pltpu.sync_copy(onest, acc.at[dstt], add=True)
        plsc.subcore_barrier()
        pltpu.sync_copy(
            acc.at[pl.ds(r0, RPT)], out.at[cid].at[pl.ds(r0, RPT), pl.ds(0, C)]
        )

    return deg


_deg16 = _make_deg()
_agg2 = _make_agg(2)
_agg1 = _make_agg(1)

_BM = 2000
_GRID = (N // _BM,)


def _prep_body(degp_ref, x_ref, w_ref, dinv_ref, hs_ref):
    deg = degp_ref[0, :, 0] + degp_ref[1, :, 0] + 1.0
    dv = lax.rsqrt(deg)[:, None]
    dinv_ref[...] = dv
    hs_ref[...] = jnp.dot(x_ref[...], w_ref[...], preferred_element_type=jnp.float32) * dv


def _mid_body(p_ref, hs_ref, dinv_ref, b_ref, w_ref, o_ref):
    dv = dinv_ref[...]
    t = (p_ref[0] + p_ref[1] + hs_ref[...]) * dv + b_ref[...]
    a = jnp.maximum(t, 0.0)
    res = jnp.dot(a, w_ref[...], preferred_element_type=jnp.float32)
    if res.shape[1] == 128:
        o_ref[...] = res * dv
    else:
        o_ref[:, :OUT_C] = res * dv
        o_ref[:, OUT_C:] = jnp.zeros_like(res)


def _final_body(p_ref, hs_ref, dinv_ref, b_ref, o_ref):
    t = (
        (p_ref[0, :, :OUT_C] + p_ref[1, :, :OUT_C] + hs_ref[:, :OUT_C])
        * dinv_ref[...]
        + b_ref[...]
    )
    m = jnp.max(t, axis=1, keepdims=True)
    lse = jnp.log(jnp.sum(jnp.exp(t - m), axis=1, keepdims=True)) + m
    o_ref[...] = t - lse


def _row_spec(c):
    return pl.BlockSpec((_BM, c), lambda i: (i, 0))


def _p_spec(c):
    return pl.BlockSpec((NC, _BM, c), lambda i: (0, i, 0))


def _full_spec(a, b):
    return pl.BlockSpec((a, b), lambda i: (0, 0))


_prep = pl.pallas_call(
    _prep_body,
    grid=_GRID,
    in_specs=[_p_spec(128), _row_spec(IN_C), _full_spec(IN_C, HID_C)],
    out_specs=[_row_spec(1), _row_spec(HID_C)],
    out_shape=[
        jax.ShapeDtypeStruct((N, 1), jnp.float32),
        jax.ShapeDtypeStruct((N, HID_C), jnp.float32),
    ],
)


def _mid(cout):
    return pl.pallas_call(
        _mid_body,
        grid=_GRID,
        in_specs=[
            _p_spec(128),
            _row_spec(128),
            _row_spec(1),
            _full_spec(1, 128),
            _full_spec(128, cout),
        ],
        out_specs=_row_spec(128),
        out_shape=jax.ShapeDtypeStruct((N, 128), jnp.float32),
    )


_mid2 = _mid(HID_C)
_mid3 = _mid(OUT_C)

_final = pl.pallas_call(
    _final_body,
    grid=_GRID,
    in_specs=[_p_spec(128), _row_spec(128), _row_spec(1), _full_spec(1, OUT_C)],
    out_specs=_row_spec(OUT_C),
    out_shape=jax.ShapeDtypeStruct((N, OUT_C), jnp.float32),
)


def kernel(x, edge_index, W1, b1, W2, b2, W3, b3):
    edge = edge_index.astype(jnp.int32)
    degp = _deg16(edge)
    dinv, hs1 = _prep(degp, x, W1)
    p1 = _agg2(hs1, edge)
    hs2 = _mid2(p1, hs1, dinv, b1.reshape(1, -1), W2)
    p2 = _agg2(hs2, edge)
    hs3 = _mid3(p2, hs2, dinv, b2.reshape(1, -1), W3)
    p3 = _agg1(hs3, edge)
    return _final(p3, hs3, dinv, b3.reshape(1, -1))
